# Initial kernel scaffold; baseline (speedup 1.0000x reference)
import jax
import jax.numpy as jnp
from jax.experimental import pallas as pl


def kernel(inputs, W_in, rows, cols, vals, bias):
    raise NotImplementedError("write your pallas kernel here")


# speedup vs baseline: 1.0012x; 1.0012x over previous
import jax
import jax.numpy as jnp
from jax.experimental import pallas as pl


def _xproj_body(x_ref, w_ref, o_ref):
    o_ref[...] = jnp.dot(x_ref[...], w_ref[...],
                         preferred_element_type=jnp.float32)


def kernel(inputs, W_in, rows, cols, vals, bias):
    B, T, DIN = inputs.shape
    GATE = W_in.shape[1]
    N = GATE // 4
    BT = B * T
    NBLK = 2048

    x2 = inputs.reshape(BT, DIN)
    xproj = pl.pallas_call(
        _xproj_body,
        grid=(GATE // NBLK,),
        in_specs=[
            pl.BlockSpec((BT, DIN), lambda j: (0, 0)),
            pl.BlockSpec((DIN, NBLK), lambda j: (0, j)),
        ],
        out_specs=pl.BlockSpec((BT, NBLK), lambda j: (0, j)),
        out_shape=jax.ShapeDtypeStruct((BT, GATE), jnp.float32),
    )(x2, W_in)
    xproj = (xproj + bias).reshape(B, T, GATE)

    def step(carry, xt):
        h, c = carry
        gathered = h[:, rows] * vals
        sp = jnp.zeros((B, GATE), jnp.float32).at[:, cols].add(gathered)
        gates = xt + sp
        i, f, g, o = jnp.split(gates, 4, axis=-1)
        c_new = jax.nn.sigmoid(f) * c + jax.nn.sigmoid(i) * jnp.tanh(g)
        h_new = jax.nn.sigmoid(o) * jnp.tanh(c_new)
        return (h_new, c_new), h_new

    h0 = jnp.zeros((B, N), jnp.float32)
    c0 = jnp.zeros((B, N), jnp.float32)
    _, hs = jax.lax.scan(step, (h0, c0), jnp.swapaxes(xproj, 0, 1))
    return jnp.swapaxes(hs, 0, 1)


# speedup vs baseline: 14.3220x; 14.3045x over previous
import functools

import jax
import jax.numpy as jnp
from jax import lax
from jax.experimental import pallas as pl
from jax.experimental.pallas import tpu as pltpu
from jax.experimental.pallas import tpu_sc as plsc

LANES = 16
NSC = 2
NTILE = 16
CH = 1024


def _xproj_body(x_ref, w_ref, o_ref):
    o_ref[...] = jnp.dot(x_ref[...], w_ref[...],
                         preferred_element_type=jnp.float32)


def _sc_body(ns, xp, rs_h, vs_h, c_h, zz, out,
             h_local, acc, y, rs_v, vs_v, c_v, cstate, hnew,
             gates_sh, h_sh):
    N, GATE, T, NCH = ns
    CPT = GATE // NTILE
    NPT = N // NTILE
    bh = lax.axis_index("c")
    w = lax.axis_index("s")

    pltpu.sync_copy(zz, h_local)
    pltpu.sync_copy(zz.at[pl.ds(0, NPT)], cstate)

    def step(t, _):
        pltpu.sync_copy(xp.at[bh, t, pl.ds(w * CPT, CPT)],
                        gates_sh.at[pl.ds(w * CPT, CPT)])
        plsc.subcore_barrier()

        @pl.when(t > 0)
        def _spmm():
            def chunk(k, _):
                pltpu.sync_copy(rs_h.at[w, k], rs_v)
                pltpu.sync_copy(vs_h.at[w, k], vs_v)
                pltpu.sync_copy(c_h.at[w, k], c_v)

                def blk(jb, _):
                    base = jb * LANES
                    rv = rs_v[pl.ds(base, LANES)]
                    vv = vs_v[pl.ds(base, LANES)]
                    for e in range(LANES):
                        y[base + e] = h_local[rv[e]] * vv[e]
                    return 0
                lax.fori_loop(0, CH // LANES, blk, 0)
                pltpu.sync_copy(y, gates_sh.at[c_v], add=True)
                return 0
            lax.fori_loop(0, NCH, chunk, 0)

        plsc.subcore_barrier()
        pltpu.sync_copy(gates_sh.at[pl.ds(w * CPT, CPT)], acc)

        def pw(nn, _):
            ig = acc[nn]
            fg = acc[NPT + nn]
            gg = acc[2 * NPT + nn]
            og = acc[3 * NPT + nn]
            c0 = cstate[nn]
            si = 1.0 / (1.0 + jnp.exp(-ig))
            sf = 1.0 / (1.0 + jnp.exp(-fg))
            so = 1.0 / (1.0 + jnp.exp(-og))
            tg = 2.0 / (1.0 + jnp.exp(-2.0 * gg)) - 1.0
            cn = sf * c0 + si * tg
            tcn = 2.0 / (1.0 + jnp.exp(-2.0 * cn)) - 1.0
            cstate[nn] = cn
            hnew[nn] = so * tcn
            return 0
        lax.fori_loop(0, NPT, pw, 0)

        pltpu.sync_copy(hnew, h_sh.at[pl.ds(w * NPT, NPT)])
        pltpu.sync_copy(hnew, out.at[bh, t, pl.ds(w * NPT, NPT)])
        plsc.subcore_barrier()
        pltpu.sync_copy(h_sh, h_local)
        plsc.subcore_barrier()
        return 0

    lax.fori_loop(0, T, step, 0)


def kernel(inputs, W_in, rows, cols, vals, bias):
    B, T, DIN = inputs.shape
    GATE = W_in.shape[1]
    N = GATE // 4
    NNZ = rows.shape[0]
    BT = B * T
    NBLK = 2048
    CPT = GATE // NTILE
    NPT = N // NTILE

    per_tile = -(-NNZ // (NTILE * CH)) * CH
    NCH = per_tile // CH
    CAP = NTILE * per_tile

    cp = ((cols >> 8) & (NTILE - 1)) * CPT + (cols >> 12) * NPT + (cols & (NPT - 1))
    g2 = jnp.arange(GATE, dtype=jnp.int32)
    src_col = (g2 & 1023) // NPT * N + (g2 >> 10) * NPT + (g2 & (NPT - 1))
    W_p = W_in[:, src_col]
    bias_p = bias[src_col]

    pad = CAP - NNZ
    rs_p = jnp.concatenate([rows, jnp.zeros((pad,), jnp.int32)])
    vs_p = jnp.concatenate([vals, jnp.zeros((pad,), jnp.float32)])
    cp_p = jnp.concatenate([cp, jnp.zeros((pad,), jnp.int32)])
    rs_h = rs_p.reshape(NTILE, NCH, CH)
    vs_h = vs_p.reshape(NTILE, NCH, CH)
    c_h = cp_p.reshape(NTILE, NCH, CH)

    x2 = inputs.reshape(BT, DIN)
    xproj = pl.pallas_call(
        _xproj_body,
        grid=(GATE // NBLK,),
        in_specs=[
            pl.BlockSpec((BT, DIN), lambda j: (0, 0)),
            pl.BlockSpec((DIN, NBLK), lambda j: (0, j)),
        ],
        out_specs=pl.BlockSpec((BT, NBLK), lambda j: (0, j)),
        out_shape=jax.ShapeDtypeStruct((BT, GATE), jnp.float32),
    )(x2, W_p)
    xproj = xproj + bias_p
    xp = xproj.reshape(NSC, LANES, T, GATE).transpose(0, 2, 3, 1)

    zz = jnp.zeros((N, LANES), jnp.float32)

    mesh = plsc.VectorSubcoreMesh(core_axis_name="c", subcore_axis_name="s")
    sc = pl.kernel(
        functools.partial(_sc_body, (N, GATE, T, NCH)),
        out_type=jax.ShapeDtypeStruct((NSC, T, N, LANES), jnp.float32),
        mesh=mesh,
        compiler_params=pltpu.CompilerParams(use_tc_tiling_on_sc=False),
        scratch_types=[
            pltpu.VMEM((N, LANES), jnp.float32),
            pltpu.VMEM((CPT, LANES), jnp.float32),
            pltpu.VMEM((CH, LANES), jnp.float32),
            pltpu.VMEM((CH,), jnp.int32),
            pltpu.VMEM((CH,), jnp.float32),
            pltpu.VMEM((CH,), jnp.int32),
            pltpu.VMEM((NPT, LANES), jnp.float32),
            pltpu.VMEM((NPT, LANES), jnp.float32),
            pltpu.VMEM_SHARED((GATE, LANES), jnp.float32),
            pltpu.VMEM_SHARED((N, LANES), jnp.float32),
        ],
    )
    out = sc(xp, rs_h, vs_h, c_h, zz)
    return out.transpose(0, 3, 1, 2).reshape(B, T, N)
